# baseline (speedup 1.0000x reference)
import functools

import jax
import jax.numpy as jnp
from jax import lax
from jax.experimental import pallas as pl
from jax.experimental.pallas import tpu as pltpu
from jax.experimental.pallas import tpu_sc as plsc

_BATCH = 16384
_D = 64
_L = 16

_info = plsc.get_sparse_core_info()
_NC, _NS = _info.num_cores, _info.num_subcores
_NW = _NC * _NS
_BPW = _BATCH // _NW
_G = _BPW // _L
_FRPW = _BPW * _D // 128


@functools.partial(
    pl.kernel,
    mesh=plsc.VectorSubcoreMesh(core_axis_name="c", subcore_axis_name="s"),
    out_type=jax.ShapeDtypeStruct((_NW, _L), jnp.float32),
    scratch_types=[
        pltpu.VMEM((_BPW,), jnp.int32),
        pltpu.VMEM((_BPW,), jnp.int32),
        pltpu.VMEM((_FRPW, 128), jnp.float32),
        pltpu.VMEM((_BPW, 128), jnp.float32),
        pltpu.VMEM((_L,), jnp.float32),
        pltpu.SemaphoreType.DMA,
        pltpu.SemaphoreType.DMA,
    ],
    compiler_params=pltpu.CompilerParams(
        use_tc_tiling_on_sc=False, needs_layout_passes=False),
)
def _center_loss_sc(features_hbm, labels_hbm, centers_hbm, out_hbm,
                    lab_v, pidx_v, feat_v, rows_v, acc_v, gsem, fsem):
    wid = lax.axis_index("s") * _NC + lax.axis_index("c")
    base = wid * _BPW

    fcopy = pltpu.async_copy(
        features_hbm.at[pl.ds(wid * _FRPW, _FRPW)], feat_v, fsem)
    pltpu.sync_copy(labels_hbm.at[pl.ds(base, _BPW)], lab_v)

    def pbody(g, _):
        v = lab_v[pl.ds(g * _L, _L)]
        pidx_v[pl.ds(g * _L, _L)] = jnp.right_shift(v, 1)
        return 0

    lax.fori_loop(0, _G, pbody, 0)
    gcopy = pltpu.async_copy(centers_hbm.at[pidx_v], rows_v, gsem)
    fcopy.wait()
    gcopy.wait()

    iota = lax.iota(jnp.int32, _L)
    fcol0 = (iota & 1) * _D
    frow0 = jnp.right_shift(iota, 1)
    zero = jnp.zeros((_L,), jnp.float32)

    def body(g, accs):
        lab16 = lab_v[pl.ds(g * _L, _L)]
        hcol0 = (lab16 & 1) * _D
        crow = g * _L + iota
        frow = g * 8 + frow0
        a = list(accs)
        for r in range(_D):
            c = plsc.load_gather(rows_v, [crow, hcol0 + r])
            f = plsc.load_gather(feat_v, [frow, fcol0 + r])
            d = f - c
            a[r % 4] = a[r % 4] + d * d
        return tuple(a)

    accs = lax.fori_loop(0, _G, body, (zero,) * 4)
    acc_v[...] = (accs[0] + accs[1]) + (accs[2] + accs[3])
    pltpu.sync_copy(acc_v, out_hbm.at[wid])


def kernel(features, labels, centers):
    partials = _center_loss_sc(
        features.reshape(_BATCH * _D // 128, 128),
        labels.astype(jnp.int32),
        centers.reshape(-1, 128),
    )
    return jnp.sum(partials) * (0.5 / _BATCH)


# speedup vs baseline: 1.3865x; 1.3865x over previous
import functools

import jax
import jax.numpy as jnp
from jax import lax
from jax.experimental import pallas as pl
from jax.experimental.pallas import tpu as pltpu
from jax.experimental.pallas import tpu_sc as plsc

_BATCH = 16384
_D = 64
_L = 16

_info = plsc.get_sparse_core_info()
_NC, _NS = _info.num_cores, _info.num_subcores
_NW = _NC * _NS
_BPW = _BATCH // _NW
_G = _BPW // _L
_CH = _D // _L
_GPW = _BPW * _CH


@functools.partial(
    pl.kernel,
    mesh=plsc.VectorSubcoreMesh(core_axis_name="c", subcore_axis_name="s"),
    out_type=jax.ShapeDtypeStruct((_NW, _L), jnp.float32),
    scratch_types=[
        pltpu.VMEM((_BPW,), jnp.int32),
        pltpu.VMEM((_GPW,), jnp.int32),
        pltpu.VMEM((_GPW, _L), jnp.float32),
        pltpu.VMEM((_GPW, _L), jnp.float32),
        pltpu.VMEM((_L,), jnp.float32),
        pltpu.SemaphoreType.DMA,
        pltpu.SemaphoreType.DMA,
    ],
    compiler_params=pltpu.CompilerParams(
        use_tc_tiling_on_sc=False, needs_layout_passes=False),
)
def _center_loss_sc(features_hbm, labels_hbm, centers_hbm, out_hbm,
                    lab_v, gidx_v, feat_v, rows_v, acc_v, gsem, fsem):
    wid = lax.axis_index("s") * _NC + lax.axis_index("c")
    base = wid * _BPW

    fcopy = pltpu.async_copy(
        features_hbm.at[pl.ds(wid * _GPW, _GPW)], feat_v, fsem)
    pltpu.sync_copy(labels_hbm.at[pl.ds(base, _BPW)], lab_v)

    iota = lax.iota(jnp.int32, _L)

    def pbody(g, _):
        l4 = lab_v[pl.ds(g * _L, _L)] * 4
        pos = (g * _L + iota) * _CH
        for c in range(_CH):
            plsc.store_scatter(gidx_v, [pos + c], l4 + c)
        return 0

    lax.fori_loop(0, _G, pbody, 0)
    gcopy = pltpu.async_copy(centers_hbm.at[gidx_v], rows_v, gsem)
    fcopy.wait()
    gcopy.wait()

    zero = jnp.zeros((_L,), jnp.float32)

    def body(i, accs):
        out = []
        for c in range(_CH):
            d = feat_v[i * _CH + c, :] - rows_v[i * _CH + c, :]
            out.append(accs[c] + d * d)
        return tuple(out)

    accs = lax.fori_loop(0, _BPW, body, (zero,) * _CH)
    acc_v[...] = (accs[0] + accs[1]) + (accs[2] + accs[3])
    pltpu.sync_copy(acc_v, out_hbm.at[wid])


def kernel(features, labels, centers):
    partials = _center_loss_sc(
        features.reshape(_BATCH * _D // _L, _L),
        labels.astype(jnp.int32),
        centers.reshape(-1, _L),
    )
    return jnp.sum(partials) * (0.5 / _BATCH)
